# baseline (speedup 1.0000x reference)
import functools

import jax
import jax.numpy as jnp
from jax import lax
from jax.experimental import pallas as pl
from jax.experimental.pallas import tpu as pltpu
from jax.experimental.pallas import tpu_sc as plsc

NC = 2
NS = 16
NW = NC * NS
LAG = 16


@functools.lru_cache(maxsize=None)
def _make_lookup(V, D, B):
    assert B % NW == 0
    b_per_w = B // NW
    mesh = plsc.VectorSubcoreMesh(core_axis_name="c", subcore_axis_name="s")

    @functools.partial(
        pl.kernel,
        mesh=mesh,
        out_type=jax.ShapeDtypeStruct((B, D), jnp.float32),
        scratch_types=[
            pltpu.VMEM((b_per_w,), jnp.int32),
            pltpu.SemaphoreType.DMA,
        ],
    )
    def lookup(table_hbm, idx_hbm, out_hbm, idx_v, sem):
        wid = lax.axis_index("s") * NC + lax.axis_index("c")
        base = wid * b_per_w
        pltpu.sync_copy(idx_hbm.at[pl.ds(base, b_per_w)], idx_v)

        def drain16(j0):
            for b in range(16):
                pltpu.make_async_copy(
                    table_hbm.at[pl.ds(0, 1)],
                    out_hbm.at[pl.ds(base + j0 + b, 1)],
                    sem,
                ).wait()

        def group(g, _):
            j0 = g * 16
            ivec = idx_v[pl.ds(j0, 16)]
            for b in range(16):
                pltpu.async_copy(
                    table_hbm.at[pl.ds(ivec[b], 1)],
                    out_hbm.at[pl.ds(base + j0 + b, 1)],
                    sem,
                )

            @pl.when(g >= LAG)
            def _():
                drain16((g - LAG) * 16)

            return ()

        n16 = b_per_w // 16
        lax.fori_loop(0, n16, group, ())

        def tail(g, _):
            drain16(g * 16)
            return ()

        lax.fori_loop(n16 - LAG, n16, tail, ())

    return lookup


def kernel(table, idx):
    idx32 = jnp.squeeze(idx, axis=-1).astype(jnp.int32)
    B = idx32.shape[0]
    V, D = table.shape
    return _make_lookup(V, D, B)(table, idx32)


# speedup vs baseline: 1.7916x; 1.7916x over previous
import functools

import jax
import jax.numpy as jnp
from jax import lax
from jax.experimental import pallas as pl
from jax.experimental.pallas import tpu as pltpu
from jax.experimental.pallas import tpu_sc as plsc

NC = 2
NS = 16
NW = NC * NS
LAG = 16


@functools.lru_cache(maxsize=None)
def _make_lookup(V, D, B):
    assert B % NW == 0
    b_per_w = B // NW
    mesh = plsc.VectorSubcoreMesh(core_axis_name="c", subcore_axis_name="s")

    @functools.partial(
        pl.kernel,
        mesh=mesh,
        out_type=jax.ShapeDtypeStruct((B, D), jnp.float32),
        scratch_types=[
            pltpu.VMEM((b_per_w,), jnp.int32),
            pltpu.VMEM((b_per_w, D), jnp.float32),
            pltpu.SemaphoreType.DMA,
        ],
    )
    def lookup(table_hbm, idx_hbm, out_hbm, idx_v, rows_v, sem):
        wid = lax.axis_index("s") * NC + lax.axis_index("c")
        base = wid * b_per_w
        pltpu.sync_copy(idx_hbm.at[pl.ds(base, b_per_w)], idx_v)

        def drain16(j0):
            for b in range(16):
                pltpu.make_async_copy(
                    table_hbm.at[pl.ds(0, 1)],
                    rows_v.at[pl.ds(j0 + b, 1)],
                    sem,
                ).wait()

        def group(g, _):
            j0 = g * 16
            ivec = idx_v[pl.ds(j0, 16)]
            for b in range(16):
                pltpu.async_copy(
                    table_hbm.at[pl.ds(ivec[b], 1)],
                    rows_v.at[pl.ds(j0 + b, 1)],
                    sem,
                )

            @pl.when(g >= LAG)
            def _():
                drain16((g - LAG) * 16)

            return ()

        n16 = b_per_w // 16
        lax.fori_loop(0, n16, group, ())

        def tail(g, _):
            drain16(g * 16)
            return ()

        lax.fori_loop(n16 - LAG, n16, tail, ())
        pltpu.sync_copy(rows_v, out_hbm.at[pl.ds(base, b_per_w)])

    return lookup


def kernel(table, idx):
    idx32 = jnp.squeeze(idx, axis=-1).astype(jnp.int32)
    B = idx32.shape[0]
    V, D = table.shape
    return _make_lookup(V, D, B)(table, idx32)
